# baseline (speedup 1.0000x reference)
import functools

import jax
import jax.numpy as jnp
from jax import lax
from jax.experimental import pallas as pl
from jax.experimental.pallas import tpu as pltpu
from jax.experimental.pallas import tpu_sc as plsc

_N = 10000
_C = 21
_K = 16
_ITERS = 3
_NPAD = 10240
_CPAD = 32
_RB = 512
_BANDS = _NPAD // _RB

_NW = 32
_ROWS_PER_W = _NPAD // _NW
_CH = 8
_NCHUNK = _ROWS_PER_W // _CH


def _knn_body(a_ref, bt_ref, idx_ref):
    a = a_ref[...]
    half = _NPAD // 2
    keys = []
    for h in range(2):
        d2 = jnp.dot(a, bt_ref[:, h * half:(h + 1) * half],
                     preferred_element_type=jnp.float32)
        col = lax.broadcasted_iota(jnp.int32, (_RB, half), 1) + h * half
        keys.append(lax.bitcast_convert_type(
            (lax.bitcast_convert_type(d2, jnp.int32)
             & jnp.int32(~0x3FFF)) | col, jnp.float32))
    s = jnp.minimum(keys[0], keys[1])
    s = jnp.minimum(s[:, :2560], s[:, 2560:])
    a1, b1 = s[:, :1280], s[:, 1280:]
    m1 = jnp.minimum(a1, b1)
    m2 = jnp.maximum(a1, b1)
    a1, b1 = m1[:, :640], m1[:, 640:]
    a2, b2 = m2[:, :640], m2[:, 640:]
    m1 = jnp.minimum(a1, b1)
    m2 = jnp.minimum(jnp.maximum(a1, b1), jnp.minimum(a2, b2))
    acc1, acc2 = m1[:, :128], m2[:, :128]
    for k in range(1, 5):
        n1, n2 = m1[:, k * 128:(k + 1) * 128], m2[:, k * 128:(k + 1) * 128]
        acc2 = jnp.minimum(jnp.maximum(acc1, n1), jnp.minimum(acc2, n2))
        acc1 = jnp.minimum(acc1, n1)
    m1, m2 = acc1, acc2
    big = jnp.float32(3e38)
    for i in range(_K):
        m = jnp.min(m1, axis=1, keepdims=True)
        idx_ref[:, i:i + 1] = (lax.bitcast_convert_type(m, jnp.int32)
                               & jnp.int32(0x3FFF))
        hit = m1 == m
        m1 = jnp.where(hit, m2, m1)
        m2 = jnp.where(hit, big, m2)


def _knn(a2, bt2):
    return pl.pallas_call(
        _knn_body,
        grid=(_BANDS,),
        in_specs=[
            pl.BlockSpec((_RB, 8), lambda i: (i, 0)),
            pl.BlockSpec((8, _NPAD), lambda i: (0, 0)),
        ],
        out_specs=pl.BlockSpec((_RB, _K), lambda i: (i, 0)),
        out_shape=jax.ShapeDtypeStruct((_NPAD, _K), jnp.int32),
    )(a2, bt2)


def _masked_softmax(x):
    lane = lax.broadcasted_iota(jnp.int32, x.shape, 1)
    valid = lane < _C
    xm = jnp.where(valid, x, -jnp.inf)
    m = jnp.max(xm, axis=1, keepdims=True)
    e = jnp.where(valid, jnp.exp(x - m), 0.0)
    s = jnp.sum(e, axis=1, keepdims=True)
    return e / s


def _softmax_body(x_ref, q_ref):
    q_ref[...] = _masked_softmax(x_ref[...])


def _softmax(logits_pad):
    return pl.pallas_call(
        _softmax_body,
        out_shape=jax.ShapeDtypeStruct((_NPAD, _CPAD), jnp.float32),
    )(logits_pad)


def _crf_sc_body(qtab, idx_hbm, ltab, wt_hbm, q_out, r_out,
                 idx_v, lg_v, wt_v, rows2, qc_v, rc_v, sem0, sem1):
    wid = lax.axis_index("s") * 2 + lax.axis_index("c")
    row_base = wid * _ROWS_PER_W
    pltpu.sync_copy(idx_hbm.at[pl.ds(wid * _NCHUNK, _NCHUNK)], idx_v)
    pltpu.sync_copy(ltab.at[pl.ds(row_base, _ROWS_PER_W)], lg_v)
    pltpu.sync_copy(wt_hbm, wt_v)
    sems = (sem0, sem1)
    lane = lax.broadcasted_iota(jnp.int32, (16,), 0)
    hi_valid = lane < (_C - 16)
    mask_f = jnp.where(hi_valid, 1.0, 0.0).astype(jnp.float32)
    negbias = jnp.where(hi_valid, 0.0, -3e38).astype(jnp.float32)

    def process(ci, b):
        for r in range(_CH):
            gi = r * _K
            a0 = rows2[b, gi, 0:16]
            a1 = rows2[b, gi, 16:32]
            for j in range(1, _K):
                a0 = a0 + rows2[b, gi + j, 0:16]
                a1 = a1 + rows2[b, gi + j, 16:32]
            rl = ci * _CH + r
            r0 = lg_v[rl, 0:16]
            r1 = lg_v[rl, 16:32]
            for m in range(_C):
                s = a0[m] if m < 16 else a1[m - 16]
                r0 = r0 + s * wt_v[m, 0:16]
                r1 = r1 + s * wt_v[m, 16:32]
            x1 = r1 * mask_f + negbias
            mx = jnp.maximum(jnp.max(r0), jnp.max(x1))
            e0 = jnp.exp(r0 - mx)
            e1 = jnp.exp(x1 - mx) * mask_f
            ssum = jnp.sum(e0) + jnp.sum(e1)
            inv = jnp.full((16,), 1.0, jnp.float32) / ssum
            qc_v[r, 0:16] = e0 * inv
            qc_v[r, 16:32] = e1 * inv
            rc_v[r, 0:16] = r0
            rc_v[r, 16:32] = r1 * mask_f
        row0 = row_base + ci * _CH
        pltpu.sync_copy(qc_v, q_out.at[pl.ds(row0, _CH)])
        pltpu.sync_copy(rc_v, r_out.at[pl.ds(row0, _CH)])

    pltpu.async_copy(qtab.at[idx_v.at[0]], rows2.at[0], sems[0])

    def loop_body(half, carry):
        for b in range(2):
            ci = half * 2 + b

            @pl.when(ci + 1 < _NCHUNK)
            def _():
                pltpu.async_copy(qtab.at[idx_v.at[ci + 1]], rows2.at[1 - b],
                                 sems[1 - b])

            pltpu.make_async_copy(qtab.at[idx_v.at[ci]], rows2.at[b],
                                  sems[b]).wait()
            process(ci, b)
        return carry

    lax.fori_loop(0, _NCHUNK // 2, loop_body, 0)


def _crf_sc(qtab, knn_flat_rows, logits_pad, wt_s):
    mesh = plsc.VectorSubcoreMesh(core_axis_name="c", subcore_axis_name="s")
    f = pl.kernel(
        _crf_sc_body,
        out_type=(jax.ShapeDtypeStruct((_NPAD, _CPAD), jnp.float32),
                  jax.ShapeDtypeStruct((_NPAD, _CPAD), jnp.float32)),
        mesh=mesh,
        scratch_types=[
            pltpu.VMEM((_NCHUNK, _CH * _K), jnp.int32),
            pltpu.VMEM((_ROWS_PER_W, _CPAD), jnp.float32),
            pltpu.VMEM((_CPAD, _CPAD), jnp.float32),
            pltpu.VMEM((2, _CH * _K, _CPAD), jnp.float32),
            pltpu.VMEM((_CH, _CPAD), jnp.float32),
            pltpu.VMEM((_CH, _CPAD), jnp.float32),
            pltpu.SemaphoreType.DMA,
            pltpu.SemaphoreType.DMA,
        ],
        compiler_params=pltpu.CompilerParams(use_tc_tiling_on_sc=False,
                                             needs_layout_passes=False),
    )
    return f(qtab, knn_flat_rows, logits_pad, wt_s)


def kernel(logits, coords, W):
    coords_pad = jnp.pad(coords, ((0, _NPAD - _N), (0, 0)))
    sq = jnp.sum(coords_pad * coords_pad, axis=1, keepdims=True)
    ones = jnp.ones((_NPAD, 1), jnp.float32)
    zeros = jnp.zeros((_NPAD, 3), jnp.float32)
    a2 = jnp.concatenate([-2.0 * coords_pad, ones, sq + 1e-3, zeros], axis=1)
    poison = jnp.where(jnp.arange(_NPAD)[:, None] < _N, sq, 1e30)
    bt2 = jnp.concatenate([coords_pad, poison, ones, zeros], axis=1).T
    knn_idx = _knn(a2, bt2)
    knn_flat_rows = knn_idx.reshape(_NPAD * _K // 128, 128)

    logits_pad = jnp.pad(logits, ((0, _NPAD - _N), (0, _CPAD - _C)))
    wt_s = jnp.pad(W.T / _K, ((0, _CPAD - _C), (0, _CPAD - _C)))

    q = _softmax(logits_pad)
    refined = None
    for _ in range(_ITERS):
        q, refined = _crf_sc(q, knn_flat_rows, logits_pad, wt_s)
    return (refined[:_N, :_C], q[:_N, :_C])


# speedup vs baseline: 1.2989x; 1.2989x over previous
import functools

import jax
import jax.numpy as jnp
from jax import lax
from jax.experimental import pallas as pl
from jax.experimental.pallas import tpu as pltpu
from jax.experimental.pallas import tpu_sc as plsc

_N = 10000
_C = 21
_K = 16
_ITERS = 3
_NPAD = 10240
_CPAD = 32
_RB = 512
_BANDS = _NPAD // _RB

_NW = 32
_ROWS_PER_W = _NPAD // _NW
_CH = 8
_NCHUNK = _ROWS_PER_W // _CH


def _knn_body(a_ref, bt_ref, idx_ref):
    a = a_ref[...]
    half = _NPAD // 2
    keys = []
    for h in range(2):
        d2 = jnp.dot(a, bt_ref[:, h * half:(h + 1) * half],
                     preferred_element_type=jnp.float32)
        col = lax.broadcasted_iota(jnp.int32, (_RB, half), 1) + h * half
        keys.append(lax.bitcast_convert_type(
            (lax.bitcast_convert_type(d2, jnp.int32)
             & jnp.int32(~0x3FFF)) | col, jnp.float32))
    s = jnp.minimum(keys[0], keys[1])
    s = jnp.minimum(s[:, :2560], s[:, 2560:])
    a1, b1 = s[:, :1280], s[:, 1280:]
    m1 = jnp.minimum(a1, b1)
    m2 = jnp.maximum(a1, b1)
    a1, b1 = m1[:, :640], m1[:, 640:]
    a2, b2 = m2[:, :640], m2[:, 640:]
    m1 = jnp.minimum(a1, b1)
    m2 = jnp.minimum(jnp.maximum(a1, b1), jnp.minimum(a2, b2))
    acc1, acc2 = m1[:, :128], m2[:, :128]
    for k in range(1, 5):
        n1, n2 = m1[:, k * 128:(k + 1) * 128], m2[:, k * 128:(k + 1) * 128]
        acc2 = jnp.minimum(jnp.maximum(acc1, n1), jnp.minimum(acc2, n2))
        acc1 = jnp.minimum(acc1, n1)
    m1, m2 = acc1, acc2
    big = jnp.float32(3e38)
    for i in range(_K):
        m = jnp.min(m1, axis=1, keepdims=True)
        idx_ref[:, i:i + 1] = (lax.bitcast_convert_type(m, jnp.int32)
                               & jnp.int32(0x3FFF))
        hit = m1 == m
        m1 = jnp.where(hit, m2, m1)
        m2 = jnp.where(hit, big, m2)


def _knn(a2, bt2):
    return pl.pallas_call(
        _knn_body,
        grid=(_BANDS,),
        in_specs=[
            pl.BlockSpec((_RB, 8), lambda i: (i, 0)),
            pl.BlockSpec((8, _NPAD), lambda i: (0, 0)),
        ],
        out_specs=pl.BlockSpec((_RB, _K), lambda i: (i, 0)),
        out_shape=jax.ShapeDtypeStruct((_NPAD, _K), jnp.int32),
    )(a2, bt2)


def _masked_softmax(x):
    lane = lax.broadcasted_iota(jnp.int32, x.shape, 1)
    valid = lane < _C
    xm = jnp.where(valid, x, -jnp.inf)
    m = jnp.max(xm, axis=1, keepdims=True)
    e = jnp.where(valid, jnp.exp(x - m), 0.0)
    s = jnp.sum(e, axis=1, keepdims=True)
    return e / s


def _softmax_body(x_ref, q_ref):
    q_ref[...] = _masked_softmax(x_ref[...]).astype(jnp.bfloat16)


def _softmax(logits_pad):
    return pl.pallas_call(
        _softmax_body,
        out_shape=jax.ShapeDtypeStruct((_NPAD, _CPAD), jnp.bfloat16),
    )(logits_pad)


def _gather_mean_body(qtab, idx_hbm, out_hbm, idx_v, rows2, acc_v, sem0, sem1):
    wid = lax.axis_index("s") * 2 + lax.axis_index("c")
    pltpu.sync_copy(idx_hbm.at[pl.ds(wid * _NCHUNK, _NCHUNK)], idx_v)
    sems = (sem0, sem1)

    def accum(ci, b):
        for r in range(_CH):
            gi = r * _K
            a0 = rows2[b, gi, 0:32]
            for j in range(1, _K):
                a0 = a0 + rows2[b, gi + j, 0:32]
            acc_v[r, 0:32] = a0
        row0 = wid * _ROWS_PER_W + ci * _CH
        pltpu.sync_copy(acc_v, out_hbm.at[pl.ds(row0, _CH)])

    pltpu.async_copy(qtab.at[idx_v.at[0]], rows2.at[0], sems[0])

    def loop_body(half, carry):
        for b in range(2):
            ci = half * 2 + b

            @pl.when(ci + 1 < _NCHUNK)
            def _():
                pltpu.async_copy(qtab.at[idx_v.at[ci + 1]], rows2.at[1 - b],
                                 sems[1 - b])

            pltpu.make_async_copy(qtab.at[idx_v.at[ci]], rows2.at[b],
                                  sems[b]).wait()
            accum(ci, b)
        return carry

    lax.fori_loop(0, _NCHUNK // 2, loop_body, 0)


def _gather_mean(qtab, knn_flat_rows):
    mesh = plsc.VectorSubcoreMesh(core_axis_name="c", subcore_axis_name="s")
    f = pl.kernel(
        _gather_mean_body,
        out_type=jax.ShapeDtypeStruct((_NPAD, _CPAD), jnp.bfloat16),
        mesh=mesh,
        scratch_types=[
            pltpu.VMEM((_NCHUNK, _CH * _K), jnp.int32),
            pltpu.VMEM((2, _CH * _K, _CPAD), jnp.bfloat16),
            pltpu.VMEM((_CH, _CPAD), jnp.bfloat16),
            pltpu.SemaphoreType.DMA,
            pltpu.SemaphoreType.DMA,
        ],
        compiler_params=pltpu.CompilerParams(use_tc_tiling_on_sc=False),
    )
    return f(qtab, knn_flat_rows)


def _crf_body(logits_ref, msg_ref, wt_ref, ref_out, q_out, qb_out):
    msg = msg_ref[...].astype(jnp.float32)
    refined = logits_ref[...] + jnp.dot(msg, wt_ref[...],
                                        preferred_element_type=jnp.float32)
    ref_out[...] = refined
    q = _masked_softmax(refined)
    q_out[...] = q
    qb_out[...] = q.astype(jnp.bfloat16)


def _crf_step(logits_pad, msg, wt_s):
    return pl.pallas_call(
        _crf_body,
        out_shape=(
            jax.ShapeDtypeStruct((_NPAD, _CPAD), jnp.float32),
            jax.ShapeDtypeStruct((_NPAD, _CPAD), jnp.float32),
            jax.ShapeDtypeStruct((_NPAD, _CPAD), jnp.bfloat16),
        ),
    )(logits_pad, msg, wt_s)


def kernel(logits, coords, W):
    coords_pad = jnp.pad(coords, ((0, _NPAD - _N), (0, 0)))
    sq = jnp.sum(coords_pad * coords_pad, axis=1, keepdims=True)
    ones = jnp.ones((_NPAD, 1), jnp.float32)
    zeros = jnp.zeros((_NPAD, 3), jnp.float32)
    a2 = jnp.concatenate([-2.0 * coords_pad, ones, sq + 1e-3, zeros], axis=1)
    poison = jnp.where(jnp.arange(_NPAD)[:, None] < _N, sq, 1e30)
    bt2 = jnp.concatenate([coords_pad, poison, ones, zeros], axis=1).T
    knn_idx = _knn(a2, bt2)
    knn_flat_rows = knn_idx.reshape(_NPAD * _K // 128, 128)

    logits_pad = jnp.pad(logits, ((0, _NPAD - _N), (0, _CPAD - _C)))
    wt_s = jnp.pad(W.T / _K, ((0, _CPAD - _C), (0, _CPAD - _C)))

    qb = _softmax(logits_pad)
    refined = q = None
    for _ in range(_ITERS):
        msg = _gather_mean(qb, knn_flat_rows)
        refined, q, qb = _crf_step(logits_pad, msg, wt_s)
    return (refined[:_N, :_C], q[:_N, :_C])


# speedup vs baseline: 1.3718x; 1.0561x over previous
import functools

import jax
import jax.numpy as jnp
from jax import lax
from jax.experimental import pallas as pl
from jax.experimental.pallas import tpu as pltpu
from jax.experimental.pallas import tpu_sc as plsc

_N = 10000
_C = 21
_K = 16
_ITERS = 3
_NPAD = 10240
_CPAD = 32
_RB = 512
_BANDS = _NPAD // _RB

_NW = 32
_ROWS_PER_W = _NPAD // _NW
_CH = 8
_NCHUNK = _ROWS_PER_W // _CH


def _knn_body(a_ref, bt_ref, l_ref, idx_ref, q0_ref):
    q0_ref[...] = _masked_softmax(l_ref[...]).astype(jnp.bfloat16)
    a = a_ref[...]
    half = _NPAD // 2
    keys = []
    for h in range(2):
        d2 = jnp.dot(a, bt_ref[:, h * half:(h + 1) * half],
                     preferred_element_type=jnp.float32)
        col = lax.broadcasted_iota(jnp.int32, (_RB, half), 1) + h * half
        keys.append(lax.bitcast_convert_type(
            (lax.bitcast_convert_type(d2, jnp.int32)
             & jnp.int32(~0x3FFF)) | col, jnp.float32))
    s = jnp.minimum(keys[0], keys[1])
    s = jnp.minimum(s[:, :2560], s[:, 2560:])
    a1, b1 = s[:, :1280], s[:, 1280:]
    m1 = jnp.minimum(a1, b1)
    m2 = jnp.maximum(a1, b1)
    a1, b1 = m1[:, :640], m1[:, 640:]
    a2, b2 = m2[:, :640], m2[:, 640:]
    m1 = jnp.minimum(a1, b1)
    m2 = jnp.minimum(jnp.maximum(a1, b1), jnp.minimum(a2, b2))
    acc1, acc2 = m1[:, :128], m2[:, :128]
    for k in range(1, 5):
        n1, n2 = m1[:, k * 128:(k + 1) * 128], m2[:, k * 128:(k + 1) * 128]
        acc2 = jnp.minimum(jnp.maximum(acc1, n1), jnp.minimum(acc2, n2))
        acc1 = jnp.minimum(acc1, n1)
    m1, m2 = acc1, acc2
    big = jnp.float32(3e38)
    for i in range(_K):
        m = jnp.min(m1, axis=1, keepdims=True)
        idx_ref[:, i:i + 1] = (lax.bitcast_convert_type(m, jnp.int32)
                               & jnp.int32(0x3FFF))
        hit = m1 == m
        m1 = jnp.where(hit, m2, m1)
        m2 = jnp.where(hit, big, m2)


def _knn(a2, bt2, logits_pad):
    return pl.pallas_call(
        _knn_body,
        grid=(_BANDS,),
        in_specs=[
            pl.BlockSpec((_RB, 8), lambda i: (i, 0)),
            pl.BlockSpec((8, _NPAD), lambda i: (0, 0)),
            pl.BlockSpec((_RB, _CPAD), lambda i: (i, 0)),
        ],
        out_specs=(pl.BlockSpec((_RB, _K), lambda i: (i, 0)),
                   pl.BlockSpec((_RB, _CPAD), lambda i: (i, 0))),
        out_shape=(jax.ShapeDtypeStruct((_NPAD, _K), jnp.int32),
                   jax.ShapeDtypeStruct((_NPAD, _CPAD), jnp.bfloat16)),
    )(a2, bt2, logits_pad)


def _masked_softmax(x):
    lane = lax.broadcasted_iota(jnp.int32, x.shape, 1)
    valid = lane < _C
    xm = jnp.where(valid, x, -jnp.inf)
    m = jnp.max(xm, axis=1, keepdims=True)
    e = jnp.where(valid, jnp.exp(x - m), 0.0)
    s = jnp.sum(e, axis=1, keepdims=True)
    return e / s


def _gather_mean_body(qtab, idx_hbm, out_hbm, idx_v, rows2, acc_v, sem0, sem1):
    wid = lax.axis_index("s") * 2 + lax.axis_index("c")
    pltpu.sync_copy(idx_hbm.at[pl.ds(wid * _NCHUNK, _NCHUNK)], idx_v)
    sems = (sem0, sem1)

    def accum(ci, b):
        for r in range(_CH):
            gi = r * _K
            a0 = rows2[b, gi, 0:32]
            for j in range(1, _K):
                a0 = a0 + rows2[b, gi + j, 0:32]
            acc_v[r, 0:32] = a0
        row0 = wid * _ROWS_PER_W + ci * _CH
        pltpu.sync_copy(acc_v, out_hbm.at[pl.ds(row0, _CH)])

    pltpu.async_copy(qtab.at[idx_v.at[0]], rows2.at[0], sems[0])

    def loop_body(half, carry):
        for b in range(2):
            ci = half * 2 + b

            @pl.when(ci + 1 < _NCHUNK)
            def _():
                pltpu.async_copy(qtab.at[idx_v.at[ci + 1]], rows2.at[1 - b],
                                 sems[1 - b])

            pltpu.make_async_copy(qtab.at[idx_v.at[ci]], rows2.at[b],
                                  sems[b]).wait()
            accum(ci, b)
        return carry

    lax.fori_loop(0, _NCHUNK // 2, loop_body, 0)


def _gather_mean(qtab, knn_flat_rows):
    mesh = plsc.VectorSubcoreMesh(core_axis_name="c", subcore_axis_name="s")
    f = pl.kernel(
        _gather_mean_body,
        out_type=jax.ShapeDtypeStruct((_NPAD, _CPAD), jnp.bfloat16),
        mesh=mesh,
        scratch_types=[
            pltpu.VMEM((_NCHUNK, _CH * _K), jnp.int32),
            pltpu.VMEM((2, _CH * _K, _CPAD), jnp.bfloat16),
            pltpu.VMEM((_CH, _CPAD), jnp.bfloat16),
            pltpu.SemaphoreType.DMA,
            pltpu.SemaphoreType.DMA,
        ],
        compiler_params=pltpu.CompilerParams(use_tc_tiling_on_sc=False),
    )
    return f(qtab, knn_flat_rows)


def _refined(logits_ref, msg_ref, wt_ref):
    msg = msg_ref[...].astype(jnp.float32)
    return logits_ref[...] + jnp.dot(msg, wt_ref[...],
                                     preferred_element_type=jnp.float32)


def _crf_mid_body(logits_ref, msg_ref, wt_ref, qb_out):
    qb_out[...] = _masked_softmax(_refined(logits_ref, msg_ref,
                                           wt_ref)).astype(jnp.bfloat16)


def _crf_mid(logits_pad, msg, wt_s):
    return pl.pallas_call(
        _crf_mid_body,
        out_shape=jax.ShapeDtypeStruct((_NPAD, _CPAD), jnp.bfloat16),
    )(logits_pad, msg, wt_s)


def _crf_last_body(logits_ref, msg_ref, wt_ref, ref_out, q_out):
    refined = _refined(logits_ref, msg_ref, wt_ref)
    ref_out[...] = refined[:_N, :_C]
    q_out[...] = _masked_softmax(refined)[:_N, :_C]


def _crf_last(logits_pad, msg, wt_s):
    return pl.pallas_call(
        _crf_last_body,
        out_shape=(
            jax.ShapeDtypeStruct((_N, _C), jnp.float32),
            jax.ShapeDtypeStruct((_N, _C), jnp.float32),
        ),
    )(logits_pad, msg, wt_s)


def kernel(logits, coords, W):
    coords_pad = jnp.pad(coords, ((0, _NPAD - _N), (0, 0)))
    sq = jnp.sum(coords_pad * coords_pad, axis=1, keepdims=True)
    ones = jnp.ones((_NPAD, 1), jnp.float32)
    zeros = jnp.zeros((_NPAD, 3), jnp.float32)
    a2 = jnp.concatenate([-2.0 * coords_pad, ones, sq + 1e-3, zeros], axis=1)
    poison = jnp.where(jnp.arange(_NPAD)[:, None] < _N, sq, 1e30)
    bt2 = jnp.concatenate([coords_pad, poison, ones, zeros], axis=1).T
    logits_pad = jnp.pad(logits, ((0, _NPAD - _N), (0, _CPAD - _C)))
    wt_s = jnp.pad(W.T / _K, ((0, _CPAD - _C), (0, _CPAD - _C)))

    knn_idx, qb = _knn(a2, bt2, logits_pad)
    knn_flat_rows = knn_idx.reshape(_NPAD * _K // 128, 128)

    for t in range(_ITERS):
        msg = _gather_mean(qb, knn_flat_rows)
        if t < _ITERS - 1:
            qb = _crf_mid(logits_pad, msg, wt_s)
    refined, q = _crf_last(logits_pad, msg, wt_s)
    return (refined, q)


# speedup vs baseline: 1.5776x; 1.1500x over previous
import functools

import jax
import jax.numpy as jnp
from jax import lax
from jax.experimental import pallas as pl
from jax.experimental.pallas import tpu as pltpu
from jax.experimental.pallas import tpu_sc as plsc

_N = 10000
_C = 21
_K = 16
_ITERS = 3
_NPAD = 10240
_CPAD = 32
_RB = 512
_BANDS = _NPAD // _RB

_NW = 32
_ROWS_PER_W = _NPAD // _NW
_CH = 8
_NCHUNK = _ROWS_PER_W // _CH


def _knn_body(a_ref, bt_ref, l_ref, idx_ref, q0_ref):
    q0_ref[...] = _masked_softmax(l_ref[...]).astype(jnp.bfloat16)
    a = a_ref[...]
    half = _NPAD // 2
    keys = []
    for h in range(2):
        d2 = jnp.dot(a, bt_ref[:, h * half:(h + 1) * half],
                     preferred_element_type=jnp.float32)
        col = lax.broadcasted_iota(jnp.int32, (_RB, half), 1) + h * half
        keys.append(lax.bitcast_convert_type(
            (lax.bitcast_convert_type(d2, jnp.int32)
             & jnp.int32(~0x3FFF)) | col, jnp.float32))
    s = jnp.minimum(keys[0], keys[1])
    s = jnp.minimum(s[:, :2560], s[:, 2560:])
    a1, b1 = s[:, :1280], s[:, 1280:]
    m1 = jnp.minimum(a1, b1)
    m2 = jnp.maximum(a1, b1)
    a1, b1 = m1[:, :640], m1[:, 640:]
    a2, b2 = m2[:, :640], m2[:, 640:]
    m1 = jnp.minimum(a1, b1)
    m2 = jnp.minimum(jnp.maximum(a1, b1), jnp.minimum(a2, b2))
    acc1, acc2 = m1[:, :128], m2[:, :128]
    for k in range(1, 5):
        n1, n2 = m1[:, k * 128:(k + 1) * 128], m2[:, k * 128:(k + 1) * 128]
        acc2 = jnp.minimum(jnp.maximum(acc1, n1), jnp.minimum(acc2, n2))
        acc1 = jnp.minimum(acc1, n1)
    m1, m2 = acc1, acc2
    big = jnp.float32(3e38)
    for i in range(_K):
        m = jnp.min(m1, axis=1, keepdims=True)
        idx_ref[:, i:i + 1] = (lax.bitcast_convert_type(m, jnp.int32)
                               & jnp.int32(0x3FFF))
        hit = m1 == m
        m1 = jnp.where(hit, m2, m1)
        m2 = jnp.where(hit, big, m2)


def _knn(a2, bt2, logits_pad):
    return pl.pallas_call(
        _knn_body,
        grid=(_BANDS,),
        in_specs=[
            pl.BlockSpec((_RB, 8), lambda i: (i, 0)),
            pl.BlockSpec((8, _NPAD), lambda i: (0, 0)),
            pl.BlockSpec((_RB, _CPAD), lambda i: (i, 0)),
        ],
        out_specs=(pl.BlockSpec((_RB, _K), lambda i: (i, 0)),
                   pl.BlockSpec((_RB, _CPAD), lambda i: (i, 0))),
        out_shape=(jax.ShapeDtypeStruct((_NPAD, _K), jnp.int32),
                   jax.ShapeDtypeStruct((_NPAD, _CPAD), jnp.bfloat16)),
    )(a2, bt2, logits_pad)


def _masked_softmax(x):
    lane = lax.broadcasted_iota(jnp.int32, x.shape, 1)
    valid = lane < _C
    xm = jnp.where(valid, x, -jnp.inf)
    m = jnp.max(xm, axis=1, keepdims=True)
    e = jnp.where(valid, jnp.exp(x - m), 0.0)
    s = jnp.sum(e, axis=1, keepdims=True)
    return e / s


def _gather_mean_body(qtab, idx_hbm, out_hbm, qsh, idx_v, rows2, acc_v,
                      sem0, sem1):
    wid = lax.axis_index("s") * 2 + lax.axis_index("c")
    sid = lax.axis_index("s")
    rows_per_sub = _NPAD // 16
    pltpu.sync_copy(qtab.at[pl.ds(sid * rows_per_sub, rows_per_sub)],
                    qsh.at[pl.ds(sid * rows_per_sub, rows_per_sub)])
    pltpu.sync_copy(idx_hbm.at[pl.ds(wid * _NCHUNK, _NCHUNK)], idx_v)
    plsc.subcore_barrier()
    sems = (sem0, sem1)

    def accum(ci, b):
        for r in range(_CH):
            gi = r * _K
            a0 = rows2[b, gi, 0:32]
            for j in range(1, _K):
                a0 = a0 + rows2[b, gi + j, 0:32]
            acc_v[r, 0:32] = a0
        row0 = wid * _ROWS_PER_W + ci * _CH
        pltpu.sync_copy(acc_v, out_hbm.at[pl.ds(row0, _CH)])

    pltpu.async_copy(qsh.at[idx_v.at[0]], rows2.at[0], sems[0])

    def loop_body(half, carry):
        for b in range(2):
            ci = half * 2 + b

            @pl.when(ci + 1 < _NCHUNK)
            def _():
                pltpu.async_copy(qsh.at[idx_v.at[ci + 1]], rows2.at[1 - b],
                                 sems[1 - b])

            pltpu.make_async_copy(qsh.at[idx_v.at[ci]], rows2.at[b],
                                  sems[b]).wait()
            accum(ci, b)
        return carry

    lax.fori_loop(0, _NCHUNK // 2, loop_body, 0)


def _gather_mean(qtab, knn_flat_rows):
    mesh = plsc.VectorSubcoreMesh(core_axis_name="c", subcore_axis_name="s")
    f = pl.kernel(
        _gather_mean_body,
        out_type=jax.ShapeDtypeStruct((_NPAD, _CPAD), jnp.bfloat16),
        mesh=mesh,
        scratch_types=[
            pltpu.VMEM_SHARED((_NPAD, _CPAD), jnp.bfloat16),
            pltpu.VMEM((_NCHUNK, _CH * _K), jnp.int32),
            pltpu.VMEM((2, _CH * _K, _CPAD), jnp.bfloat16),
            pltpu.VMEM((_CH, _CPAD), jnp.bfloat16),
            pltpu.SemaphoreType.DMA,
            pltpu.SemaphoreType.DMA,
        ],
        compiler_params=pltpu.CompilerParams(use_tc_tiling_on_sc=False),
    )
    return f(qtab, knn_flat_rows)


def _refined(logits_ref, msg_ref, wt_ref):
    msg = msg_ref[...].astype(jnp.float32)
    return logits_ref[...] + jnp.dot(msg, wt_ref[...],
                                     preferred_element_type=jnp.float32)


def _crf_mid_body(logits_ref, msg_ref, wt_ref, qb_out):
    qb_out[...] = _masked_softmax(_refined(logits_ref, msg_ref,
                                           wt_ref)).astype(jnp.bfloat16)


def _crf_mid(logits_pad, msg, wt_s):
    return pl.pallas_call(
        _crf_mid_body,
        out_shape=jax.ShapeDtypeStruct((_NPAD, _CPAD), jnp.bfloat16),
    )(logits_pad, msg, wt_s)


def _crf_last_body(logits_ref, msg_ref, wt_ref, ref_out, q_out):
    refined = _refined(logits_ref, msg_ref, wt_ref)
    ref_out[...] = refined[:_N, :_C]
    q_out[...] = _masked_softmax(refined)[:_N, :_C]


def _crf_last(logits_pad, msg, wt_s):
    return pl.pallas_call(
        _crf_last_body,
        out_shape=(
            jax.ShapeDtypeStruct((_N, _C), jnp.float32),
            jax.ShapeDtypeStruct((_N, _C), jnp.float32),
        ),
    )(logits_pad, msg, wt_s)


def kernel(logits, coords, W):
    coords_pad = jnp.pad(coords, ((0, _NPAD - _N), (0, 0)))
    sq = jnp.sum(coords_pad * coords_pad, axis=1, keepdims=True)
    ones = jnp.ones((_NPAD, 1), jnp.float32)
    zeros = jnp.zeros((_NPAD, 3), jnp.float32)
    a2 = jnp.concatenate([-2.0 * coords_pad, ones, sq + 1e-3, zeros], axis=1)
    poison = jnp.where(jnp.arange(_NPAD)[:, None] < _N, sq, 1e30)
    bt2 = jnp.concatenate([coords_pad, poison, ones, zeros], axis=1).T
    logits_pad = jnp.pad(logits, ((0, _NPAD - _N), (0, _CPAD - _C)))
    wt_s = jnp.pad(W.T / _K, ((0, _CPAD - _C), (0, _CPAD - _C)))

    knn_idx, qb = _knn(a2, bt2, logits_pad)
    knn_flat_rows = knn_idx.reshape(_NPAD * _K // 128, 128)

    for t in range(_ITERS):
        msg = _gather_mean(qb, knn_flat_rows)
        if t < _ITERS - 1:
            qb = _crf_mid(logits_pad, msg, wt_s)
    refined, q = _crf_last(logits_pad, msg, wt_s)
    return (refined, q)
